# Initial kernel scaffold; baseline (speedup 1.0000x reference)
import jax
import jax.numpy as jnp
from jax.experimental import pallas as pl


def kernel(x, edge_index, edge_attr, W1, b1, W2, b2, W3, b3, W4, b4):
    raise NotImplementedError("write your pallas kernel here")


# speedup vs baseline: 7.0006x; 7.0006x over previous
import functools

import jax
import jax.numpy as jnp
from jax import lax
from jax.experimental import pallas as pl
from jax.experimental.pallas import tpu as pltpu
from jax.experimental.pallas import tpu_sc as plsc

N = 10000
D = 128
NC = 2
NS = 16
NW = NC * NS
Np = 10240
CH = 128
NCH = 79
EPW = CH * NCH
Ep = EPW * NW
RPT = Np // NS

_mesh = plsc.VectorSubcoreMesh(core_axis_name="c", subcore_axis_name="s")
_Z16 = None


@functools.partial(
    pl.kernel,
    out_type=jax.ShapeDtypeStruct((NC, Np), jnp.float32),
    mesh=_mesh,
    scratch_types=[
        pltpu.VMEM((CH,), jnp.int32),
        pltpu.VMEM((CH,), jnp.float32),
        pltpu.VMEM((RPT,), jnp.float32),
        pltpu.VMEM_SHARED((Np,), jnp.float32),
    ],
)
def _deg_kernel(col_hbm, w_hbm, out_hbm, cix, wv, buf, acc):
    cid = lax.axis_index("c")
    sid = lax.axis_index("s")
    wid = sid * NC + cid

    def zb(i, carry):
        buf[pl.ds(i * 16, 16)] = jnp.zeros((16,), jnp.float32)
        return carry

    lax.fori_loop(0, RPT // 16, zb, 0)
    pltpu.sync_copy(buf, acc.at[pl.ds(sid * RPT, RPT)])
    plsc.subcore_barrier()

    def chunk(g, carry):
        off = pl.multiple_of(wid * EPW + g * CH, CH)
        pltpu.sync_copy(col_hbm.at[pl.ds(off, CH)], cix)
        pltpu.sync_copy(w_hbm.at[pl.ds(off, CH)], wv)
        pltpu.sync_copy(wv, acc.at[cix], add=True)
        return carry

    lax.fori_loop(0, NCH, chunk, 0)
    plsc.subcore_barrier()
    pltpu.sync_copy(acc.at[pl.ds(sid * RPT, RPT)], buf)
    pltpu.sync_copy(buf, out_hbm.at[cid, pl.ds(sid * RPT, RPT)])


@functools.partial(
    pl.kernel,
    out_type=jax.ShapeDtypeStruct((NC, Np, D), jnp.float32),
    mesh=_mesh,
    scratch_types=[
        pltpu.VMEM((CH,), jnp.int32),
        pltpu.VMEM((CH,), jnp.int32),
        pltpu.VMEM((CH,), jnp.float32),
        pltpu.VMEM((CH, D), jnp.float32),
        pltpu.VMEM((128, D), jnp.float32),
        pltpu.VMEM_SHARED((Np, D), jnp.float32),
        pltpu.SemaphoreType.DMA,
    ],
)
def _prop_kernel(y_hbm, row_hbm, col_hbm, w_hbm, out_hbm,
                 rix, cix, wv, msg, buf, acc, sem):
    cid = lax.axis_index("c")
    sid = lax.axis_index("s")
    wid = sid * NC + cid

    def zb(i, carry):
        for f in range(D // 16):
            buf[i, pl.ds(f * 16, 16)] = jnp.zeros((16,), jnp.float32)
        return carry

    lax.fori_loop(0, 128, zb, 0)
    for k in range(RPT // 128):
        pltpu.sync_copy(buf, acc.at[pl.ds(sid * RPT + k * 128, 128), :])
    plsc.subcore_barrier()

    def chunk(g, carry):
        off = pl.multiple_of(wid * EPW + g * CH, CH)
        pltpu.sync_copy(row_hbm.at[pl.ds(off, CH)], rix)
        pltpu.sync_copy(col_hbm.at[pl.ds(off, CH)], cix)
        pltpu.sync_copy(w_hbm.at[pl.ds(off, CH)], wv)
        pltpu.async_copy(y_hbm.at[rix], msg, sem).wait()

        def qbody(q, c2):
            w16 = wv[pl.ds(q * 16, 16)]
            for j in range(16):
                e = q * 16 + j
                wb = jnp.full((16,), w16[j], jnp.float32)
                for f in range(D // 16):
                    msg[e, pl.ds(f * 16, 16)] = msg[e, pl.ds(f * 16, 16)] * wb
            return c2

        lax.fori_loop(0, CH // 16, qbody, 0)
        pltpu.sync_copy(msg, acc.at[cix], add=True)
        return carry

    lax.fori_loop(0, NCH, chunk, 0)
    plsc.subcore_barrier()
    for k in range(RPT // 128):
        pltpu.sync_copy(acc.at[pl.ds(sid * RPT + k * 128, 128), :], buf)
        pltpu.sync_copy(buf, out_hbm.at[cid, pl.ds(sid * RPT + k * 128, 128), :])


_ROWS = 1280
_GRID = Np // _ROWS


def _blk(w):
    return pl.BlockSpec((_ROWS, w), lambda i: (i, 0))


def _full(shape):
    return pl.BlockSpec(shape, lambda i: (0,) * len(shape))


def _pre1_body(deg0, deg1, x, W, dinv_o, y_o):
    deg = deg0[:] + deg1[:] + 1.0
    dinv = lax.rsqrt(deg)
    dinv_o[:] = dinv
    y_o[:] = dinv * jnp.dot(x[:], W[:], preferred_element_type=jnp.float32)


_pre1 = pl.pallas_call(
    _pre1_body,
    grid=(_GRID,),
    in_specs=[_blk(1), _blk(1), _blk(D), _full((D, D))],
    out_specs=[_blk(1), _blk(D)],
    out_shape=[jax.ShapeDtypeStruct((Np, 1), jnp.float32),
               jax.ShapeDtypeStruct((Np, D), jnp.float32)],
)


def _elu(v):
    return jnp.where(v > 0, v, jnp.exp(v) - 1.0)


def _boundary_body(s0, s1, y, dinv, b, W, y_next):
    h = _elu(dinv[:] * (s0[:] + s1[:] + y[:]) + b[:])
    y_next[:] = dinv[:] * jnp.dot(h, W[:], preferred_element_type=jnp.float32)


_boundary = pl.pallas_call(
    _boundary_body,
    grid=(_GRID,),
    in_specs=[_blk(D), _blk(D), _blk(D), _blk(1), _full((1, D)), _full((D, D))],
    out_specs=_blk(D),
    out_shape=jax.ShapeDtypeStruct((Np, D), jnp.float32),
)


def _boundary3_body(s0, s1, y, dinv, b, y_next):
    h = _elu(dinv[:] * (s0[:] + s1[:] + y[:]) + b[:])
    y_next[:] = dinv[:] * h


_boundary3 = pl.pallas_call(
    _boundary3_body,
    grid=(_GRID,),
    in_specs=[_blk(D), _blk(D), _blk(D), _blk(1), _full((1, D))],
    out_specs=_blk(D),
    out_shape=jax.ShapeDtypeStruct((Np, D), jnp.float32),
)


def _final_body(s0, s1, y, dinv, W, b, out):
    h = dinv[:] * (s0[:] + s1[:] + y[:])
    z = jnp.dot(h, W[:], preferred_element_type=jnp.float32) + b[:]
    m = jnp.max(z, axis=1, keepdims=True)
    t = z - m
    lse = jnp.log(jnp.sum(jnp.exp(t), axis=1, keepdims=True))
    out[:] = t - lse


_final = pl.pallas_call(
    _final_body,
    grid=(_GRID,),
    in_specs=[_blk(D), _blk(D), _blk(D), _blk(1), _full((D, 40)), _full((1, 40))],
    out_specs=_blk(40),
    out_shape=jax.ShapeDtypeStruct((Np, 40), jnp.float32),
)


def kernel(x, edge_index, edge_attr, W1, b1, W2, b2, W3, b3, W4, b4):
    ei = edge_index.astype(jnp.int32)
    epad = jnp.zeros((Ep - ei.shape[1],), jnp.int32)
    row = jnp.concatenate([ei[0], epad])
    col = jnp.concatenate([ei[1], epad])
    w = jnp.concatenate([edge_attr.astype(jnp.float32),
                         jnp.zeros((Ep - ei.shape[1],), jnp.float32)])
    xp = jnp.concatenate([x, jnp.zeros((Np - N, D), jnp.float32)], axis=0)

    deg = _deg_kernel(col, w)
    deg0 = deg[0].reshape(Np, 1)
    deg1 = deg[1].reshape(Np, 1)
    dinv, y = _pre1(deg0, deg1, xp, W1)

    s = _prop_kernel(y, row, col, w)
    y = _boundary(s[0], s[1], y, dinv, b1.reshape(1, D), W2)
    s = _prop_kernel(y, row, col, w)
    y = _boundary(s[0], s[1], y, dinv, b2.reshape(1, D), W3)
    s = _prop_kernel(y, row, col, w)
    y = _boundary3(s[0], s[1], y, dinv, b3.reshape(1, D))
    s = _prop_kernel(y, row, col, w)
    out = _final(s[0], s[1], y, dinv, W4, b4.reshape(1, 40))
    return out[:N]
